# baseline (speedup 1.0000x reference)
import jax
import jax.numpy as jnp
from jax import lax
from jax.experimental import pallas as pl
from jax.experimental.pallas import tpu as pltpu
from jax.experimental.pallas import tpu_sc as plsc

B, C, H, W = 8, 2, 512, 512
HW = H * W
N = B * H * W
MIN_KEPT = 100000
K_RANK = min(MIN_KEPT, N - 1)
LOGIT_T = 0.8472978603872037

H_TC = 256
ROWS = 64
GB, GH = B, H_TC // ROWS
GH_FULL = H // ROWS

NC, NS = 2, 16
NW = NC * NS
RPT = (H - H_TC) // (NW // B)
PX_TILE = RPT * W
CH = 4096
NCH = PX_TILE // CH

_LOG1P_C = (0.9999970542923676, -0.4998254710554547, 0.33078789064803327,
            -0.23417367475167797, 0.14810677481238943, -0.06577012721513113,
            0.014026852411466048)


def _dt(x_ref, t_ref):
    diff = x_ref[1] - x_ref[0]
    tt = t_ref[...]
    return jnp.where(tt == 1, diff, -diff), tt


def _loss(d, tt, w_ref):
    wt = jnp.where(tt == 1, w_ref[1], w_ref[0])
    return wt * (jnp.log1p(jnp.exp(-jnp.abs(d))) + jnp.maximum(-d, 0.0))


def _pass1_body(x_ref, t_ref, w_ref, clt_ref, sum_ref, a_lt, a_sum):
    b, h = pl.program_id(0), pl.program_id(1)
    first = jnp.logical_and(b == 0, h == 0)
    last = jnp.logical_and(b == GB - 1, h == GH - 1)

    @pl.when(first)
    def _():
        a_lt[...] = jnp.zeros_like(a_lt)
        a_sum[...] = jnp.zeros_like(a_sum)

    d, tt = _dt(x_ref, t_ref)
    L = jnp.float32(LOGIT_T)
    sel = d < L
    one = jnp.float32(1.0)
    zero = jnp.float32(0.0)
    a_lt[...] += jnp.where(sel, one, zero)
    a_sum[...] += jnp.where(sel, _loss(d, tt, w_ref), zero)

    @pl.when(last)
    def _():
        clt_ref[0, 0] = jnp.sum(a_lt[...])
        sum_ref[0, 0] = jnp.sum(a_sum[...])


def _sc_body(x_hbm, t_hbm, w0_hbm, wd_hbm, ocnt_hbm, osum_hbm,
             bx0, bx1, bt, bw0, bwd, bcnt, bsum):
    cc = lax.axis_index("c")
    ss = lax.axis_index("s")
    wid = ss * NC + cc
    b = wid // (NW // B)
    q = wid % (NW // B)
    row0 = H_TC + q * RPT
    x0_off = b * (2 * HW) + row0 * W
    t_off = b * HW + row0 * W

    pltpu.sync_copy(w0_hbm, bw0)
    pltpu.sync_copy(wd_hbm, bwd)
    w0v = bw0[...]
    wdv = bwd[...]
    L = jnp.float32(LOGIT_T)

    def chunk_body(k, carry):
        cnt, sm = carry
        off = k * CH
        pltpu.sync_copy(x_hbm.at[pl.ds(x0_off + off, CH)], bx0)
        pltpu.sync_copy(x_hbm.at[pl.ds(x0_off + HW + off, CH)], bx1)
        pltpu.sync_copy(t_hbm.at[pl.ds(t_off + off, CH)], bt)

        def inner(i, c2):
            cnt2, sm2 = c2
            s16 = pl.ds(i * 16, 16)
            x0 = bx0[s16]
            x1 = bx1[s16]
            tf = bt[s16].astype(jnp.float32)
            d = (x1 - x0) * (tf + tf - 1.0)
            sel = d < L
            y = jnp.exp(-jnp.abs(d))
            p = jnp.float32(_LOG1P_C[6])
            for cf in _LOG1P_C[5::-1]:
                p = p * y + jnp.float32(cf)
            sp = y * p + jnp.maximum(-d, jnp.float32(0.0))
            wt = w0v + tf * wdv
            cnt2 = cnt2 + jnp.where(sel, jnp.float32(1.0), jnp.float32(0.0))
            sm2 = sm2 + jnp.where(sel, wt * sp, jnp.float32(0.0))
            return cnt2, sm2

        return lax.fori_loop(0, CH // 16, inner, (cnt, sm))

    z = jnp.zeros((16,), jnp.float32)
    cnt, sm = lax.fori_loop(0, NCH, chunk_body, (z, z))
    bcnt[...] = cnt
    bsum[...] = sm
    pltpu.sync_copy(bcnt, ocnt_hbm.at[wid])
    pltpu.sync_copy(bsum, osum_hbm.at[wid])


def _key(d):
    bits = lax.bitcast_convert_type(d, jnp.int32)
    return jnp.where(bits >= 0, bits, bits ^ jnp.int32(0x7FFFFFFF))


def _bisect_body(x_ref, t_ref, w_ref, cnt_ref, sum_ref, sm_ref, sf_ref):
    j, b, h = pl.program_id(0), pl.program_id(1), pl.program_id(2)
    first = jnp.logical_and(b == 0, h == 0)

    @pl.when(jnp.logical_and(first, j == 0))
    def _():
        sm_ref[0] = jnp.int32(-2147483648)
        sm_ref[1] = 0
        sm_ref[2] = 0
        sf_ref[0] = 0.0

    @pl.when(jnp.logical_and(first, jnp.logical_and(j > 0, j <= 32)))
    def _():
        bump = jnp.where(
            sm_ref[1] <= K_RANK,
            lax.shift_left(jnp.int32(1), jnp.clip(32 - j, 0, 31)), 0)
        sm_ref[0] += bump
        sm_ref[1] = 0

    d, tt = _dt(x_ref, t_ref)
    key = _key(d)

    @pl.when(j < 32)
    def _():
        t = sm_ref[0] + lax.shift_left(jnp.int32(1), jnp.clip(31 - j, 0, 31))
        sm_ref[1] += jnp.sum((key < t).astype(jnp.int32))

    @pl.when(j == 32)
    def _():
        sel = key < sm_ref[0]
        sm_ref[2] += jnp.sum(sel.astype(jnp.int32))
        sf_ref[0] += jnp.sum(jnp.where(sel, _loss(d, tt, w_ref), 0.0))

    @pl.when(j == 33)
    def _():
        cnt_ref[0, 0] = sm_ref[2]
        sum_ref[0, 0] = sf_ref[0]


def _mk_specs(three_d):
    off = 1 if three_d else 0

    def xmap(*ids):
        return (ids[off], ids[off + 1], 0)

    def tmap(*ids):
        return (ids[off] * (GH_FULL if three_d else GH) + ids[off + 1], 0)

    return [
        pl.BlockSpec((2, ROWS, W), xmap),
        pl.BlockSpec((ROWS, W), tmap),
        pl.BlockSpec(memory_space=pltpu.SMEM),
    ]


def _scalar_outs(dtypes):
    return (
        tuple(jax.ShapeDtypeStruct((1, 1), dt) for dt in dtypes),
        tuple(pl.BlockSpec(memory_space=pltpu.SMEM) for _ in dtypes),
    )


@jax.jit
def kernel(predict, target, weight):
    xv = predict.reshape(B * C, H, W)
    tv = target.astype(jnp.int32).reshape(B * H, W)
    wv = weight.astype(jnp.float32)
    xf = predict.reshape(-1)
    tf = target.astype(jnp.int32).reshape(-1)
    w0f = jnp.full((16,), wv[0], jnp.float32)
    wdf = jnp.full((16,), wv[1] - wv[0], jnp.float32)

    sc_cnt, sc_sum = pl.kernel(
        _sc_body,
        out_type=(jax.ShapeDtypeStruct((NW, 16), jnp.float32),
                  jax.ShapeDtypeStruct((NW, 16), jnp.float32)),
        mesh=plsc.VectorSubcoreMesh(core_axis_name="c", subcore_axis_name="s"),
        scratch_types=[pltpu.VMEM((CH,), jnp.float32),
                       pltpu.VMEM((CH,), jnp.float32),
                       pltpu.VMEM((CH,), jnp.int32),
                       pltpu.VMEM((16,), jnp.float32),
                       pltpu.VMEM((16,), jnp.float32),
                       pltpu.VMEM((16,), jnp.float32),
                       pltpu.VMEM((16,), jnp.float32)],
    )(xf, tf, w0f, wdf)

    out_shape, out_specs = _scalar_outs((jnp.float32, jnp.float32))
    clt, s_lt = pl.pallas_call(
        _pass1_body,
        grid=(GB, GH),
        in_specs=_mk_specs(False),
        out_specs=list(out_specs),
        out_shape=list(out_shape),
        scratch_shapes=[pltpu.VMEM((ROWS, W), jnp.float32)] * 2,
    )(xv, tv, wv)

    cnt_lt = (clt[0, 0] + jnp.sum(sc_cnt)).astype(jnp.int32)
    s_all = s_lt[0, 0] + jnp.sum(sc_sum)

    def common(_):
        return s_all, cnt_lt

    def rare(_):
        o_shape, o_specs = _scalar_outs((jnp.int32, jnp.float32))
        cnt, tot = pl.pallas_call(
            _bisect_body,
            grid=(34, GB, GH_FULL),
            in_specs=_mk_specs(True),
            out_specs=list(o_specs),
            out_shape=list(o_shape),
            scratch_shapes=[pltpu.SMEM((3,), jnp.int32),
                            pltpu.SMEM((1,), jnp.float32)],
        )(xv, tv, wv)
        return tot[0, 0], cnt[0, 0]

    total, cnt = lax.cond(cnt_lt >= K_RANK + 1, common, rare, operand=None)
    return jnp.where(cnt == 0, total,
                     total / jnp.maximum(cnt, 1).astype(jnp.float32))


# speedup vs baseline: 1.5395x; 1.5395x over previous
import jax
import jax.numpy as jnp
from jax import lax
from jax.experimental import pallas as pl
from jax.experimental.pallas import tpu as pltpu
from jax.experimental.pallas import tpu_sc as plsc

B, C, H, W = 8, 2, 512, 512
HW = H * W
N = B * H * W
MIN_KEPT = 100000
K_RANK = min(MIN_KEPT, N - 1)
LOGIT_T = 0.8472978603872037

H_TC = 256
ROWS = 64
GB, GH = B, H_TC // ROWS
GH_FULL = H // ROWS

NC, NS = 2, 16
NW = NC * NS
RPT = (H - H_TC) // (NW // B)
CHR = 16
NCH = RPT // CHR
UNR = 8

_LOG1P_C = (0.9999970542923676, -0.4998254710554547, 0.33078789064803327,
            -0.23417367475167797, 0.14810677481238943, -0.06577012721513113,
            0.014026852411466048)


def _dt(x_ref, t_ref):
    diff = x_ref[1] - x_ref[0]
    tt = t_ref[...]
    return jnp.where(tt == 1, diff, -diff), tt


def _loss(d, tt, w_ref):
    wt = jnp.where(tt == 1, w_ref[1], w_ref[0])
    return wt * (jnp.log1p(jnp.exp(-jnp.abs(d))) + jnp.maximum(-d, 0.0))


def _pass1_body(x_ref, t_ref, w_ref, clt_ref, sum_ref, a_lt, a_sum):
    b, h = pl.program_id(0), pl.program_id(1)
    first = jnp.logical_and(b == 0, h == 0)
    last = jnp.logical_and(b == GB - 1, h == GH - 1)

    @pl.when(first)
    def _():
        a_lt[...] = jnp.zeros_like(a_lt)
        a_sum[...] = jnp.zeros_like(a_sum)

    d, tt = _dt(x_ref, t_ref)
    L = jnp.float32(LOGIT_T)
    sel = d < L
    one = jnp.float32(1.0)
    zero = jnp.float32(0.0)
    a_lt[...] += jnp.where(sel, one, zero)
    a_sum[...] += jnp.where(sel, _loss(d, tt, w_ref), zero)

    @pl.when(last)
    def _():
        clt_ref[0, 0] = jnp.sum(a_lt[...])
        sum_ref[0, 0] = jnp.sum(a_sum[...])


def _sc_body(x_hbm, t_hbm, w0_hbm, wd_hbm, ocnt_hbm, osum_hbm,
             bx0, bx1, bt, bw0, bwd, bcnt, bsum):
    cc = lax.axis_index("c")
    ss = lax.axis_index("s")
    wid = ss * NC + cc
    b = wid // (NW // B)
    q = wid % (NW // B)
    row0 = H_TC + q * RPT

    pltpu.sync_copy(w0_hbm, bw0)
    pltpu.sync_copy(wd_hbm, bwd)
    w0v = bw0[...]
    wdv = bwd[...]
    L = jnp.float32(LOGIT_T)

    def chunk_body(k, carry):
        cnt, sm = carry
        r = row0 + k * CHR
        pltpu.sync_copy(x_hbm.at[2 * b, pl.ds(r, CHR), :], bx0)
        pltpu.sync_copy(x_hbm.at[2 * b + 1, pl.ds(r, CHR), :], bx1)
        pltpu.sync_copy(t_hbm.at[pl.ds(b * H + r, CHR), :], bt)

        def inner(i, c2):
            cnt2, sm2 = c2
            s16 = pl.ds(i * 16, 16)
            for u in range(CHR):
                x0 = bx0[u, s16]
                x1 = bx1[u, s16]
                tf = bt[u, s16].astype(jnp.float32)
                d = (x1 - x0) * (tf + tf - 1.0)
                sel = d < L
                y = jnp.exp(-jnp.abs(d))
                p = jnp.float32(_LOG1P_C[6])
                for cf in _LOG1P_C[5::-1]:
                    p = p * y + jnp.float32(cf)
                sp = y * p + jnp.maximum(-d, jnp.float32(0.0))
                wt = w0v + tf * wdv
                cnt2 = cnt2 + jnp.where(sel, jnp.float32(1.0),
                                        jnp.float32(0.0))
                sm2 = sm2 + jnp.where(sel, wt * sp, jnp.float32(0.0))
            return cnt2, sm2

        return lax.fori_loop(0, W // 16, inner, (cnt, sm))

    z = jnp.zeros((16,), jnp.float32)
    cnt, sm = lax.fori_loop(0, NCH, chunk_body, (z, z))
    bcnt[...] = cnt
    bsum[...] = sm
    pltpu.sync_copy(bcnt, ocnt_hbm.at[wid])
    pltpu.sync_copy(bsum, osum_hbm.at[wid])


def _key(d):
    bits = lax.bitcast_convert_type(d, jnp.int32)
    return jnp.where(bits >= 0, bits, bits ^ jnp.int32(0x7FFFFFFF))


def _bisect_body(x_ref, t_ref, w_ref, cnt_ref, sum_ref, sm_ref, sf_ref):
    j, b, h = pl.program_id(0), pl.program_id(1), pl.program_id(2)
    first = jnp.logical_and(b == 0, h == 0)

    @pl.when(jnp.logical_and(first, j == 0))
    def _():
        sm_ref[0] = jnp.int32(-2147483648)
        sm_ref[1] = 0
        sm_ref[2] = 0
        sf_ref[0] = 0.0

    @pl.when(jnp.logical_and(first, jnp.logical_and(j > 0, j <= 32)))
    def _():
        bump = jnp.where(
            sm_ref[1] <= K_RANK,
            lax.shift_left(jnp.int32(1), jnp.clip(32 - j, 0, 31)), 0)
        sm_ref[0] += bump
        sm_ref[1] = 0

    d, tt = _dt(x_ref, t_ref)
    key = _key(d)

    @pl.when(j < 32)
    def _():
        t = sm_ref[0] + lax.shift_left(jnp.int32(1), jnp.clip(31 - j, 0, 31))
        sm_ref[1] += jnp.sum((key < t).astype(jnp.int32))

    @pl.when(j == 32)
    def _():
        sel = key < sm_ref[0]
        sm_ref[2] += jnp.sum(sel.astype(jnp.int32))
        sf_ref[0] += jnp.sum(jnp.where(sel, _loss(d, tt, w_ref), 0.0))

    @pl.when(j == 33)
    def _():
        cnt_ref[0, 0] = sm_ref[2]
        sum_ref[0, 0] = sf_ref[0]


def _mk_specs(three_d):
    off = 1 if three_d else 0

    def xmap(*ids):
        return (ids[off], ids[off + 1], 0)

    def tmap(*ids):
        return (ids[off] * (GH_FULL if three_d else GH) + ids[off + 1], 0)

    return [
        pl.BlockSpec((2, ROWS, W), xmap),
        pl.BlockSpec((ROWS, W), tmap),
        pl.BlockSpec(memory_space=pltpu.SMEM),
    ]


def _scalar_outs(dtypes):
    return (
        tuple(jax.ShapeDtypeStruct((1, 1), dt) for dt in dtypes),
        tuple(pl.BlockSpec(memory_space=pltpu.SMEM) for _ in dtypes),
    )


@jax.jit
def kernel(predict, target, weight):
    xv = predict.reshape(B * C, H, W)
    tv = target.astype(jnp.int32).reshape(B * H, W)
    wv = weight.astype(jnp.float32)
    w0f = jnp.full((16,), wv[0], jnp.float32)
    wdf = jnp.full((16,), wv[1] - wv[0], jnp.float32)

    sc_cnt, sc_sum = pl.kernel(
        _sc_body,
        out_type=(jax.ShapeDtypeStruct((NW, 16), jnp.float32),
                  jax.ShapeDtypeStruct((NW, 16), jnp.float32)),
        mesh=plsc.VectorSubcoreMesh(core_axis_name="c", subcore_axis_name="s"),
        scratch_types=[pltpu.VMEM((CHR, W), jnp.float32),
                       pltpu.VMEM((CHR, W), jnp.float32),
                       pltpu.VMEM((CHR, W), jnp.int32),
                       pltpu.VMEM((16,), jnp.float32),
                       pltpu.VMEM((16,), jnp.float32),
                       pltpu.VMEM((16,), jnp.float32),
                       pltpu.VMEM((16,), jnp.float32)],
    )(xv, tv, w0f, wdf)

    out_shape, out_specs = _scalar_outs((jnp.float32, jnp.float32))
    clt, s_lt = pl.pallas_call(
        _pass1_body,
        grid=(GB, GH),
        in_specs=_mk_specs(False),
        out_specs=list(out_specs),
        out_shape=list(out_shape),
        scratch_shapes=[pltpu.VMEM((ROWS, W), jnp.float32)] * 2,
    )(xv, tv, wv)

    cnt_lt = (clt[0, 0] + jnp.sum(sc_cnt)).astype(jnp.int32)
    s_all = s_lt[0, 0] + jnp.sum(sc_sum)

    def common(_):
        return s_all, cnt_lt

    def rare(_):
        o_shape, o_specs = _scalar_outs((jnp.int32, jnp.float32))
        cnt, tot = pl.pallas_call(
            _bisect_body,
            grid=(34, GB, GH_FULL),
            in_specs=_mk_specs(True),
            out_specs=list(o_specs),
            out_shape=list(o_shape),
            scratch_shapes=[pltpu.SMEM((3,), jnp.int32),
                            pltpu.SMEM((1,), jnp.float32)],
        )(xv, tv, wv)
        return tot[0, 0], cnt[0, 0]

    total, cnt = lax.cond(cnt_lt >= K_RANK + 1, common, rare, operand=None)
    return jnp.where(cnt == 0, total,
                     total / jnp.maximum(cnt, 1).astype(jnp.float32))


# speedup vs baseline: 1.7519x; 1.1380x over previous
import jax
import jax.numpy as jnp
from jax import lax
from jax.experimental import pallas as pl
from jax.experimental.pallas import tpu as pltpu
from jax.experimental.pallas import tpu_sc as plsc

B, C, H, W = 8, 2, 512, 512
HW = H * W
N = B * H * W
MIN_KEPT = 100000
K_RANK = min(MIN_KEPT, N - 1)
LOGIT_T = 0.8472978603872037

H_TC = 256
ROWS = 64
GB, GH = B, H_TC // ROWS
GH_FULL = H // ROWS

NC, NS = 2, 16
NW = NC * NS
RPT = (H - H_TC) // (NW // B)
CHR = 16
NCH = RPT // CHR
UNR = 8

_LOG1P_C = (0.9999970542923676, -0.4998254710554547, 0.33078789064803327,
            -0.23417367475167797, 0.14810677481238943, -0.06577012721513113,
            0.014026852411466048)


def _dt(x_ref, t_ref):
    diff = x_ref[1] - x_ref[0]
    tt = t_ref[...]
    return jnp.where(tt == 1, diff, -diff), tt


def _loss(d, tt, w_ref):
    wt = jnp.where(tt == 1, w_ref[1], w_ref[0])
    return wt * (jnp.log1p(jnp.exp(-jnp.abs(d))) + jnp.maximum(-d, 0.0))


def _pass1_body(x_ref, t_ref, w_ref, clt_ref, sum_ref, a_lt, a_sum):
    b, h = pl.program_id(0), pl.program_id(1)
    first = jnp.logical_and(b == 0, h == 0)
    last = jnp.logical_and(b == GB - 1, h == GH - 1)

    @pl.when(first)
    def _():
        a_lt[...] = jnp.zeros_like(a_lt)
        a_sum[...] = jnp.zeros_like(a_sum)

    d, tt = _dt(x_ref, t_ref)
    L = jnp.float32(LOGIT_T)
    sel = d < L
    one = jnp.float32(1.0)
    zero = jnp.float32(0.0)
    a_lt[...] += jnp.where(sel, one, zero)
    a_sum[...] += jnp.where(sel, _loss(d, tt, w_ref), zero)

    @pl.when(last)
    def _():
        clt_ref[0, 0] = jnp.sum(a_lt[...])
        sum_ref[0, 0] = jnp.sum(a_sum[...])


def _sc_body(x_hbm, t_hbm, w0_hbm, wd_hbm, ocnt_hbm, osum_hbm,
             bx0, bx1, bt, bw0, bwd, bcnt, bsum, sem0, sem1):
    cc = lax.axis_index("c")
    ss = lax.axis_index("s")
    wid = ss * NC + cc
    b = wid // (NW // B)
    q = wid % (NW // B)
    row0 = H_TC + q * RPT

    pltpu.sync_copy(w0_hbm, bw0)
    pltpu.sync_copy(wd_hbm, bwd)
    w0v = bw0[...]
    wdv = bwd[...]
    L = jnp.float32(LOGIT_T)
    sems = (sem0, sem1)

    def issue(k):
        r = row0 + k * CHR
        p = k % 2
        s = sems[p]
        return (
            pltpu.async_copy(x_hbm.at[2 * b, pl.ds(r, CHR), :], bx0.at[p], s),
            pltpu.async_copy(x_hbm.at[2 * b + 1, pl.ds(r, CHR), :],
                             bx1.at[p], s),
            pltpu.async_copy(t_hbm.at[pl.ds(b * H + r, CHR), :], bt.at[p], s),
        )

    def compute(p, cnt, sm):
        def inner(i, c2):
            cnt2, sm2 = c2
            s16 = pl.ds(i * 16, 16)
            for u in range(CHR):
                x0 = bx0[p, u, s16]
                x1 = bx1[p, u, s16]
                tf = bt[p, u, s16].astype(jnp.float32)
                d = (x1 - x0) * (tf + tf - 1.0)
                sel = d < L
                y = jnp.exp(-jnp.abs(d))
                q = jnp.float32(_LOG1P_C[6])
                for cf in _LOG1P_C[5::-1]:
                    q = q * y + jnp.float32(cf)
                sp = y * q + jnp.maximum(-d, jnp.float32(0.0))
                wt = w0v + tf * wdv
                cnt2 = cnt2 + jnp.where(sel, jnp.float32(1.0),
                                        jnp.float32(0.0))
                sm2 = sm2 + jnp.where(sel, wt * sp, jnp.float32(0.0))
            return cnt2, sm2

        return lax.fori_loop(0, W // 16, inner, (cnt, sm))

    cnt = jnp.zeros((16,), jnp.float32)
    sm = jnp.zeros((16,), jnp.float32)
    pending = issue(0)
    for k in range(NCH):
        nxt = issue(k + 1) if k + 1 < NCH else None
        for h in pending:
            h.wait()
        cnt, sm = compute(k % 2, cnt, sm)
        pending = nxt
    bcnt[...] = cnt
    bsum[...] = sm
    pltpu.sync_copy(bcnt, ocnt_hbm.at[wid])
    pltpu.sync_copy(bsum, osum_hbm.at[wid])


def _key(d):
    bits = lax.bitcast_convert_type(d, jnp.int32)
    return jnp.where(bits >= 0, bits, bits ^ jnp.int32(0x7FFFFFFF))


def _bisect_body(x_ref, t_ref, w_ref, cnt_ref, sum_ref, sm_ref, sf_ref):
    j, b, h = pl.program_id(0), pl.program_id(1), pl.program_id(2)
    first = jnp.logical_and(b == 0, h == 0)

    @pl.when(jnp.logical_and(first, j == 0))
    def _():
        sm_ref[0] = jnp.int32(-2147483648)
        sm_ref[1] = 0
        sm_ref[2] = 0
        sf_ref[0] = 0.0

    @pl.when(jnp.logical_and(first, jnp.logical_and(j > 0, j <= 32)))
    def _():
        bump = jnp.where(
            sm_ref[1] <= K_RANK,
            lax.shift_left(jnp.int32(1), jnp.clip(32 - j, 0, 31)), 0)
        sm_ref[0] += bump
        sm_ref[1] = 0

    d, tt = _dt(x_ref, t_ref)
    key = _key(d)

    @pl.when(j < 32)
    def _():
        t = sm_ref[0] + lax.shift_left(jnp.int32(1), jnp.clip(31 - j, 0, 31))
        sm_ref[1] += jnp.sum((key < t).astype(jnp.int32))

    @pl.when(j == 32)
    def _():
        sel = key < sm_ref[0]
        sm_ref[2] += jnp.sum(sel.astype(jnp.int32))
        sf_ref[0] += jnp.sum(jnp.where(sel, _loss(d, tt, w_ref), 0.0))

    @pl.when(j == 33)
    def _():
        cnt_ref[0, 0] = sm_ref[2]
        sum_ref[0, 0] = sf_ref[0]


def _mk_specs(three_d):
    off = 1 if three_d else 0

    def xmap(*ids):
        return (ids[off], ids[off + 1], 0)

    def tmap(*ids):
        return (ids[off] * (GH_FULL if three_d else GH) + ids[off + 1], 0)

    return [
        pl.BlockSpec((2, ROWS, W), xmap),
        pl.BlockSpec((ROWS, W), tmap),
        pl.BlockSpec(memory_space=pltpu.SMEM),
    ]


def _scalar_outs(dtypes):
    return (
        tuple(jax.ShapeDtypeStruct((1, 1), dt) for dt in dtypes),
        tuple(pl.BlockSpec(memory_space=pltpu.SMEM) for _ in dtypes),
    )


@jax.jit
def kernel(predict, target, weight):
    xv = predict.reshape(B * C, H, W)
    tv = target.astype(jnp.int32).reshape(B * H, W)
    wv = weight.astype(jnp.float32)
    w0f = jnp.full((16,), wv[0], jnp.float32)
    wdf = jnp.full((16,), wv[1] - wv[0], jnp.float32)

    sc_cnt, sc_sum = pl.kernel(
        _sc_body,
        out_type=(jax.ShapeDtypeStruct((NW, 16), jnp.float32),
                  jax.ShapeDtypeStruct((NW, 16), jnp.float32)),
        mesh=plsc.VectorSubcoreMesh(core_axis_name="c", subcore_axis_name="s"),
        scratch_types=[pltpu.VMEM((2, CHR, W), jnp.float32),
                       pltpu.VMEM((2, CHR, W), jnp.float32),
                       pltpu.VMEM((2, CHR, W), jnp.int32),
                       pltpu.VMEM((16,), jnp.float32),
                       pltpu.VMEM((16,), jnp.float32),
                       pltpu.VMEM((16,), jnp.float32),
                       pltpu.VMEM((16,), jnp.float32),
                       pltpu.SemaphoreType.DMA,
                       pltpu.SemaphoreType.DMA],
    )(xv, tv, w0f, wdf)

    out_shape, out_specs = _scalar_outs((jnp.float32, jnp.float32))
    clt, s_lt = pl.pallas_call(
        _pass1_body,
        grid=(GB, GH),
        in_specs=_mk_specs(False),
        out_specs=list(out_specs),
        out_shape=list(out_shape),
        scratch_shapes=[pltpu.VMEM((ROWS, W), jnp.float32)] * 2,
    )(xv, tv, wv)

    cnt_lt = (clt[0, 0] + jnp.sum(sc_cnt)).astype(jnp.int32)
    s_all = s_lt[0, 0] + jnp.sum(sc_sum)

    def common(_):
        return s_all, cnt_lt

    def rare(_):
        o_shape, o_specs = _scalar_outs((jnp.int32, jnp.float32))
        cnt, tot = pl.pallas_call(
            _bisect_body,
            grid=(34, GB, GH_FULL),
            in_specs=_mk_specs(True),
            out_specs=list(o_specs),
            out_shape=list(o_shape),
            scratch_shapes=[pltpu.SMEM((3,), jnp.int32),
                            pltpu.SMEM((1,), jnp.float32)],
        )(xv, tv, wv)
        return tot[0, 0], cnt[0, 0]

    total, cnt = lax.cond(cnt_lt >= K_RANK + 1, common, rare, operand=None)
    return jnp.where(cnt == 0, total,
                     total / jnp.maximum(cnt, 1).astype(jnp.float32))


# speedup vs baseline: 1.8169x; 1.0371x over previous
import jax
import jax.numpy as jnp
from jax import lax
from jax.experimental import pallas as pl
from jax.experimental.pallas import tpu as pltpu
from jax.experimental.pallas import tpu_sc as plsc

B, C, H, W = 8, 2, 512, 512
HW = H * W
N = B * H * W
MIN_KEPT = 100000
K_RANK = min(MIN_KEPT, N - 1)
LOGIT_T = 0.8472978603872037

H_TC = 384
ROWS = 128
GB, GH = B, H_TC // ROWS
GH_FULL = H // ROWS

NC, NS = 2, 16
NW = NC * NS
RPT = (H - H_TC) // (NW // B)
CHR = 16
NCH = RPT // CHR
UNR = 8

_LOG1P_C = (0.9999970542923676, -0.4998254710554547, 0.33078789064803327,
            -0.23417367475167797, 0.14810677481238943, -0.06577012721513113,
            0.014026852411466048)


def _dt(x_ref, t_ref):
    diff = x_ref[1] - x_ref[0]
    tt = t_ref[...]
    return jnp.where(tt == 1, diff, -diff), tt


def _loss(d, tt, w_ref):
    wt = jnp.where(tt == 1, w_ref[1], w_ref[0])
    return wt * (jnp.log1p(jnp.exp(-jnp.abs(d))) + jnp.maximum(-d, 0.0))


def _pass1_body(x_ref, t_ref, w_ref, clt_ref, sum_ref, a_lt, a_sum):
    b, h = pl.program_id(0), pl.program_id(1)
    first = jnp.logical_and(b == 0, h == 0)
    last = jnp.logical_and(b == GB - 1, h == GH - 1)

    @pl.when(first)
    def _():
        a_lt[...] = jnp.zeros_like(a_lt)
        a_sum[...] = jnp.zeros_like(a_sum)

    d, tt = _dt(x_ref, t_ref)
    L = jnp.float32(LOGIT_T)
    sel = d < L
    one = jnp.float32(1.0)
    zero = jnp.float32(0.0)
    a_lt[...] += jnp.where(sel, one, zero)
    a_sum[...] += jnp.where(sel, _loss(d, tt, w_ref), zero)

    @pl.when(last)
    def _():
        clt_ref[0, 0] = jnp.sum(a_lt[...])
        sum_ref[0, 0] = jnp.sum(a_sum[...])


def _sc_body(x_hbm, t_hbm, w0_hbm, wd_hbm, ocnt_hbm, osum_hbm,
             bx0, bx1, bt, bw0, bwd, bcnt, bsum, sem0, sem1):
    cc = lax.axis_index("c")
    ss = lax.axis_index("s")
    wid = ss * NC + cc
    b = wid // (NW // B)
    q = wid % (NW // B)
    row0 = H_TC + q * RPT

    pltpu.sync_copy(w0_hbm, bw0)
    pltpu.sync_copy(wd_hbm, bwd)
    w0v = bw0[...]
    wdv = bwd[...]
    L = jnp.float32(LOGIT_T)
    sems = (sem0, sem1)

    def issue(k):
        r = row0 + k * CHR
        p = k % 2
        s = sems[p]
        return (
            pltpu.async_copy(x_hbm.at[2 * b, pl.ds(r, CHR), :], bx0.at[p], s),
            pltpu.async_copy(x_hbm.at[2 * b + 1, pl.ds(r, CHR), :],
                             bx1.at[p], s),
            pltpu.async_copy(t_hbm.at[pl.ds(b * H + r, CHR), :], bt.at[p], s),
        )

    def compute(p, cnt, sm):
        def inner(i, c2):
            cnt2, sm2 = c2
            s16 = pl.ds(i * 16, 16)
            for u in range(CHR):
                x0 = bx0[p, u, s16]
                x1 = bx1[p, u, s16]
                tf = bt[p, u, s16].astype(jnp.float32)
                d = (x1 - x0) * (tf + tf - 1.0)
                sel = d < L
                y = jnp.exp(-jnp.abs(d))
                q = jnp.float32(_LOG1P_C[6])
                for cf in _LOG1P_C[5::-1]:
                    q = q * y + jnp.float32(cf)
                sp = y * q + jnp.maximum(-d, jnp.float32(0.0))
                wt = w0v + tf * wdv
                cnt2 = cnt2 + jnp.where(sel, jnp.float32(1.0),
                                        jnp.float32(0.0))
                sm2 = sm2 + jnp.where(sel, wt * sp, jnp.float32(0.0))
            return cnt2, sm2

        return lax.fori_loop(0, W // 16, inner, (cnt, sm))

    cnt = jnp.zeros((16,), jnp.float32)
    sm = jnp.zeros((16,), jnp.float32)
    pending = issue(0)
    for k in range(NCH):
        nxt = issue(k + 1) if k + 1 < NCH else None
        for h in pending:
            h.wait()
        cnt, sm = compute(k % 2, cnt, sm)
        pending = nxt
    bcnt[...] = cnt
    bsum[...] = sm
    pltpu.sync_copy(bcnt, ocnt_hbm.at[wid])
    pltpu.sync_copy(bsum, osum_hbm.at[wid])


def _key(d):
    bits = lax.bitcast_convert_type(d, jnp.int32)
    return jnp.where(bits >= 0, bits, bits ^ jnp.int32(0x7FFFFFFF))


def _bisect_body(x_ref, t_ref, w_ref, cnt_ref, sum_ref, sm_ref, sf_ref):
    j, b, h = pl.program_id(0), pl.program_id(1), pl.program_id(2)
    first = jnp.logical_and(b == 0, h == 0)

    @pl.when(jnp.logical_and(first, j == 0))
    def _():
        sm_ref[0] = jnp.int32(-2147483648)
        sm_ref[1] = 0
        sm_ref[2] = 0
        sf_ref[0] = 0.0

    @pl.when(jnp.logical_and(first, jnp.logical_and(j > 0, j <= 32)))
    def _():
        bump = jnp.where(
            sm_ref[1] <= K_RANK,
            lax.shift_left(jnp.int32(1), jnp.clip(32 - j, 0, 31)), 0)
        sm_ref[0] += bump
        sm_ref[1] = 0

    d, tt = _dt(x_ref, t_ref)
    key = _key(d)

    @pl.when(j < 32)
    def _():
        t = sm_ref[0] + lax.shift_left(jnp.int32(1), jnp.clip(31 - j, 0, 31))
        sm_ref[1] += jnp.sum((key < t).astype(jnp.int32))

    @pl.when(j == 32)
    def _():
        sel = key < sm_ref[0]
        sm_ref[2] += jnp.sum(sel.astype(jnp.int32))
        sf_ref[0] += jnp.sum(jnp.where(sel, _loss(d, tt, w_ref), 0.0))

    @pl.when(j == 33)
    def _():
        cnt_ref[0, 0] = sm_ref[2]
        sum_ref[0, 0] = sf_ref[0]


def _mk_specs(three_d):
    off = 1 if three_d else 0

    def xmap(*ids):
        return (ids[off], ids[off + 1], 0)

    def tmap(*ids):
        return (ids[off] * (GH_FULL if three_d else GH) + ids[off + 1], 0)

    return [
        pl.BlockSpec((2, ROWS, W), xmap),
        pl.BlockSpec((ROWS, W), tmap),
        pl.BlockSpec(memory_space=pltpu.SMEM),
    ]


def _scalar_outs(dtypes):
    return (
        tuple(jax.ShapeDtypeStruct((1, 1), dt) for dt in dtypes),
        tuple(pl.BlockSpec(memory_space=pltpu.SMEM) for _ in dtypes),
    )


@jax.jit
def kernel(predict, target, weight):
    xv = predict.reshape(B * C, H, W)
    tv = target.astype(jnp.int32).reshape(B * H, W)
    wv = weight.astype(jnp.float32)
    w0f = jnp.full((16,), wv[0], jnp.float32)
    wdf = jnp.full((16,), wv[1] - wv[0], jnp.float32)

    sc_cnt, sc_sum = pl.kernel(
        _sc_body,
        out_type=(jax.ShapeDtypeStruct((NW, 16), jnp.float32),
                  jax.ShapeDtypeStruct((NW, 16), jnp.float32)),
        mesh=plsc.VectorSubcoreMesh(core_axis_name="c", subcore_axis_name="s"),
        scratch_types=[pltpu.VMEM((2, CHR, W), jnp.float32),
                       pltpu.VMEM((2, CHR, W), jnp.float32),
                       pltpu.VMEM((2, CHR, W), jnp.int32),
                       pltpu.VMEM((16,), jnp.float32),
                       pltpu.VMEM((16,), jnp.float32),
                       pltpu.VMEM((16,), jnp.float32),
                       pltpu.VMEM((16,), jnp.float32),
                       pltpu.SemaphoreType.DMA,
                       pltpu.SemaphoreType.DMA],
    )(xv, tv, w0f, wdf)

    out_shape, out_specs = _scalar_outs((jnp.float32, jnp.float32))
    clt, s_lt = pl.pallas_call(
        _pass1_body,
        grid=(GB, GH),
        in_specs=_mk_specs(False),
        out_specs=list(out_specs),
        out_shape=list(out_shape),
        scratch_shapes=[pltpu.VMEM((ROWS, W), jnp.float32)] * 2,
    )(xv, tv, wv)

    cnt_lt = (clt[0, 0] + jnp.sum(sc_cnt)).astype(jnp.int32)
    s_all = s_lt[0, 0] + jnp.sum(sc_sum)

    def common(_):
        return s_all, cnt_lt

    def rare(_):
        o_shape, o_specs = _scalar_outs((jnp.int32, jnp.float32))
        cnt, tot = pl.pallas_call(
            _bisect_body,
            grid=(34, GB, GH_FULL),
            in_specs=_mk_specs(True),
            out_specs=list(o_specs),
            out_shape=list(o_shape),
            scratch_shapes=[pltpu.SMEM((3,), jnp.int32),
                            pltpu.SMEM((1,), jnp.float32)],
        )(xv, tv, wv)
        return tot[0, 0], cnt[0, 0]

    total, cnt = lax.cond(cnt_lt >= K_RANK + 1, common, rare, operand=None)
    return jnp.where(cnt == 0, total,
                     total / jnp.maximum(cnt, 1).astype(jnp.float32))


# speedup vs baseline: 1.9177x; 1.0555x over previous
import jax
import jax.numpy as jnp
from jax import lax
from jax.experimental import pallas as pl
from jax.experimental.pallas import tpu as pltpu
from jax.experimental.pallas import tpu_sc as plsc

B, C, H, W = 8, 2, 512, 512
HW = H * W
N = B * H * W
MIN_KEPT = 100000
K_RANK = min(MIN_KEPT, N - 1)
LOGIT_T = 0.8472978603872037

H_TC = 384
ROWS = 128
GB, GH = B, H_TC // ROWS
GH_FULL = H // ROWS

NC, NS = 2, 16
NW = NC * NS
RPT = (H - H_TC) // (NW // B)
CHR = 8
NCH = RPT // CHR

_LOG1P_C = (0.9999970542923676, -0.4998254710554547, 0.33078789064803327,
            -0.23417367475167797, 0.14810677481238943, -0.06577012721513113,
            0.014026852411466048)


def _dt(x_ref, t_ref):
    diff = x_ref[1] - x_ref[0]
    tt = t_ref[...]
    return jnp.where(tt == 1, diff, -diff), tt


def _loss(d, tt, w_ref):
    wt = jnp.where(tt == 1, w_ref[1], w_ref[0])
    return wt * (jnp.log1p(jnp.exp(-jnp.abs(d))) + jnp.maximum(-d, 0.0))


def _pass1_body(x_ref, t_ref, w_ref, clt_ref, sum_ref, a_lt, a_sum):
    b, h = pl.program_id(0), pl.program_id(1)
    first = jnp.logical_and(b == 0, h == 0)
    last = jnp.logical_and(b == GB - 1, h == GH - 1)

    @pl.when(first)
    def _():
        a_lt[...] = jnp.zeros_like(a_lt)
        a_sum[...] = jnp.zeros_like(a_sum)

    d, tt = _dt(x_ref, t_ref)
    L = jnp.float32(LOGIT_T)
    sel = d < L
    one = jnp.float32(1.0)
    zero = jnp.float32(0.0)
    a_lt[...] += jnp.where(sel, one, zero)
    a_sum[...] += jnp.where(sel, _loss(d, tt, w_ref), zero)

    @pl.when(last)
    def _():
        clt_ref[0, 0] = jnp.sum(a_lt[...])
        sum_ref[0, 0] = jnp.sum(a_sum[...])


def _sc_body(x_hbm, t_hbm, oacc_hbm, bx0, bx1, bt, bacc, sem0, sem1):
    cc = lax.axis_index("c")
    ss = lax.axis_index("s")
    wid = ss * NC + cc
    b = wid // (NW // B)
    q = wid % (NW // B)
    row0 = H_TC + q * RPT

    L = jnp.float32(LOGIT_T)
    sems = (sem0, sem1)

    def issue(k):
        r = row0 + k * CHR
        p = k % 2
        s = sems[p]
        return (
            pltpu.async_copy(x_hbm.at[2 * b, pl.ds(r, CHR), :], bx0.at[p], s),
            pltpu.async_copy(x_hbm.at[2 * b + 1, pl.ds(r, CHR), :],
                             bx1.at[p], s),
            pltpu.async_copy(t_hbm.at[pl.ds(b * H + r, CHR), :], bt.at[p], s),
        )

    def compute(p, carry):
        def inner(i, c2):
            cnt2, sp2, s12 = c2
            s16 = pl.ds(i * 16, 16)
            for u in range(CHR):
                x0 = bx0[p, u, s16]
                x1 = bx1[p, u, s16]
                tf = bt[p, u, s16].astype(jnp.float32)
                d = (x1 - x0) * (tf + tf - 1.0)
                ind = jnp.where(d < L, jnp.float32(1.0), jnp.float32(0.0))
                y = jnp.exp(-jnp.abs(d))
                qq = jnp.float32(_LOG1P_C[6])
                for cf in _LOG1P_C[5::-1]:
                    qq = qq * y + jnp.float32(cf)
                spv = (y * qq + jnp.maximum(-d, jnp.float32(0.0))) * ind
                cnt2 = cnt2 + ind
                sp2 = sp2 + spv
                s12 = s12 + spv * tf
            return cnt2, sp2, s12

        return lax.fori_loop(0, W // 16, inner, carry)

    z = jnp.zeros((16,), jnp.float32)
    carry = (z, z, z)
    pending = issue(0)
    for k in range(NCH):
        nxt = issue(k + 1) if k + 1 < NCH else None
        for h in pending:
            h.wait()
        carry = compute(k % 2, carry)
        pending = nxt
    bacc[0] = carry[0]
    bacc[1] = carry[1]
    bacc[2] = carry[2]
    pltpu.sync_copy(bacc, oacc_hbm.at[wid])


def _key(d):
    bits = lax.bitcast_convert_type(d, jnp.int32)
    return jnp.where(bits >= 0, bits, bits ^ jnp.int32(0x7FFFFFFF))


def _bisect_body(x_ref, t_ref, w_ref, cnt_ref, sum_ref, sm_ref, sf_ref):
    j, b, h = pl.program_id(0), pl.program_id(1), pl.program_id(2)
    first = jnp.logical_and(b == 0, h == 0)

    @pl.when(jnp.logical_and(first, j == 0))
    def _():
        sm_ref[0] = jnp.int32(-2147483648)
        sm_ref[1] = 0
        sm_ref[2] = 0
        sf_ref[0] = 0.0

    @pl.when(jnp.logical_and(first, jnp.logical_and(j > 0, j <= 32)))
    def _():
        bump = jnp.where(
            sm_ref[1] <= K_RANK,
            lax.shift_left(jnp.int32(1), jnp.clip(32 - j, 0, 31)), 0)
        sm_ref[0] += bump
        sm_ref[1] = 0

    d, tt = _dt(x_ref, t_ref)
    key = _key(d)

    @pl.when(j < 32)
    def _():
        t = sm_ref[0] + lax.shift_left(jnp.int32(1), jnp.clip(31 - j, 0, 31))
        sm_ref[1] += jnp.sum((key < t).astype(jnp.int32))

    @pl.when(j == 32)
    def _():
        sel = key < sm_ref[0]
        sm_ref[2] += jnp.sum(sel.astype(jnp.int32))
        sf_ref[0] += jnp.sum(jnp.where(sel, _loss(d, tt, w_ref), 0.0))

    @pl.when(j == 33)
    def _():
        cnt_ref[0, 0] = sm_ref[2]
        sum_ref[0, 0] = sf_ref[0]


def _mk_specs(three_d):
    off = 1 if three_d else 0

    def xmap(*ids):
        return (ids[off], ids[off + 1], 0)

    def tmap(*ids):
        return (ids[off] * (GH_FULL if three_d else GH) + ids[off + 1], 0)

    return [
        pl.BlockSpec((2, ROWS, W), xmap),
        pl.BlockSpec((ROWS, W), tmap),
        pl.BlockSpec(memory_space=pltpu.SMEM),
    ]


def _scalar_outs(dtypes):
    return (
        tuple(jax.ShapeDtypeStruct((1, 1), dt) for dt in dtypes),
        tuple(pl.BlockSpec(memory_space=pltpu.SMEM) for _ in dtypes),
    )


@jax.jit
def kernel(predict, target, weight):
    xv = predict.reshape(B * C, H, W)
    tv = target.astype(jnp.int32).reshape(B * H, W)
    wv = weight.astype(jnp.float32)
    sc_acc = pl.kernel(
        _sc_body,
        out_type=jax.ShapeDtypeStruct((NW, 3, 16), jnp.float32),
        mesh=plsc.VectorSubcoreMesh(core_axis_name="c", subcore_axis_name="s"),
        scratch_types=[pltpu.VMEM((2, CHR, W), jnp.float32),
                       pltpu.VMEM((2, CHR, W), jnp.float32),
                       pltpu.VMEM((2, CHR, W), jnp.int32),
                       pltpu.VMEM((3, 16), jnp.float32),
                       pltpu.SemaphoreType.DMA,
                       pltpu.SemaphoreType.DMA],
    )(xv, tv)

    out_shape, out_specs = _scalar_outs((jnp.float32, jnp.float32))
    clt, s_lt = pl.pallas_call(
        _pass1_body,
        grid=(GB, GH),
        in_specs=_mk_specs(False),
        out_specs=list(out_specs),
        out_shape=list(out_shape),
        scratch_shapes=[pltpu.VMEM((ROWS, W), jnp.float32)] * 2,
    )(xv, tv, wv)

    parts = jnp.sum(sc_acc, axis=(0, 2))
    cnt_lt = (clt[0, 0] + parts[0]).astype(jnp.int32)
    s_all = s_lt[0, 0] + wv[0] * (parts[1] - parts[2]) + wv[1] * parts[2]

    def common(_):
        return s_all, cnt_lt

    def rare(_):
        o_shape, o_specs = _scalar_outs((jnp.int32, jnp.float32))
        cnt, tot = pl.pallas_call(
            _bisect_body,
            grid=(34, GB, GH_FULL),
            in_specs=_mk_specs(True),
            out_specs=list(o_specs),
            out_shape=list(o_shape),
            scratch_shapes=[pltpu.SMEM((3,), jnp.int32),
                            pltpu.SMEM((1,), jnp.float32)],
        )(xv, tv, wv)
        return tot[0, 0], cnt[0, 0]

    total, cnt = lax.cond(cnt_lt >= K_RANK + 1, common, rare, operand=None)
    return jnp.where(cnt == 0, total,
                     total / jnp.maximum(cnt, 1).astype(jnp.float32))


# speedup vs baseline: 2.0894x; 1.0895x over previous
import jax
import jax.numpy as jnp
from jax import lax
from jax.experimental import pallas as pl
from jax.experimental.pallas import tpu as pltpu
from jax.experimental.pallas import tpu_sc as plsc

B, C, H, W = 8, 2, 512, 512
HW = H * W
N = B * H * W
MIN_KEPT = 100000
K_RANK = min(MIN_KEPT, N - 1)
LOGIT_T = 0.8472978603872037

H_TC = 384
ROWS = 192
GB, GH = B, H_TC // ROWS
ROWS_BIS = 128
GH_FULL = H // ROWS_BIS

NC, NS = 2, 16
NW = NC * NS
RPT = (H - H_TC) // (NW // B)
CHR = 8
NCH = RPT // CHR

_LOG1P_C = (0.9999970542923676, -0.4998254710554547, 0.33078789064803327,
            -0.23417367475167797, 0.14810677481238943, -0.06577012721513113,
            0.014026852411466048)


def _dt(x_ref, t_ref):
    diff = x_ref[1] - x_ref[0]
    tt = t_ref[...]
    return jnp.where(tt == 1, diff, -diff), tt


def _loss(d, tt, w_ref):
    wt = jnp.where(tt == 1, w_ref[1], w_ref[0])
    return wt * (jnp.log1p(jnp.exp(-jnp.abs(d))) + jnp.maximum(-d, 0.0))


def _pass1_body(x_ref, t_ref, w_ref, clt_ref, sum_ref, a_lt, a_sum):
    b, h = pl.program_id(0), pl.program_id(1)
    first = jnp.logical_and(b == 0, h == 0)
    last = jnp.logical_and(b == GB - 1, h == GH - 1)

    @pl.when(first)
    def _():
        a_lt[...] = jnp.zeros_like(a_lt)
        a_sum[...] = jnp.zeros_like(a_sum)

    d, tt = _dt(x_ref, t_ref)
    L = jnp.float32(LOGIT_T)
    sel = d < L
    one = jnp.float32(1.0)
    zero = jnp.float32(0.0)
    a_lt[...] += jnp.where(sel, one, zero)
    a_sum[...] += jnp.where(sel, _loss(d, tt, w_ref), zero)

    @pl.when(last)
    def _():
        clt_ref[0, 0] = jnp.sum(a_lt[...])
        sum_ref[0, 0] = jnp.sum(a_sum[...])


def _sc_body(x_hbm, t_hbm, oacc_hbm, bx0, bx1, bt, bacc, sem0, sem1):
    cc = lax.axis_index("c")
    ss = lax.axis_index("s")
    wid = ss * NC + cc
    b = wid // (NW // B)
    q = wid % (NW // B)
    row0 = H_TC + q * RPT

    L = jnp.float32(LOGIT_T)
    sems = (sem0, sem1)

    def issue(k):
        r = row0 + k * CHR
        p = k % 2
        s = sems[p]
        return (
            pltpu.async_copy(x_hbm.at[2 * b, pl.ds(r, CHR), :], bx0.at[p], s),
            pltpu.async_copy(x_hbm.at[2 * b + 1, pl.ds(r, CHR), :],
                             bx1.at[p], s),
            pltpu.async_copy(t_hbm.at[pl.ds(b * H + r, CHR), :], bt.at[p], s),
        )

    def compute(p, carry):
        def inner(i, c2):
            cnt2, sp2, s12 = c2
            s16 = pl.ds(i * 16, 16)
            for u in range(CHR):
                x0 = bx0[p, u, s16]
                x1 = bx1[p, u, s16]
                tf = bt[p, u, s16].astype(jnp.float32)
                d = (x1 - x0) * (tf + tf - 1.0)
                ind = jnp.where(d < L, jnp.float32(1.0), jnp.float32(0.0))
                y = jnp.exp(-jnp.abs(d))
                qq = jnp.float32(_LOG1P_C[6])
                for cf in _LOG1P_C[5::-1]:
                    qq = qq * y + jnp.float32(cf)
                spv = (y * qq + jnp.maximum(-d, jnp.float32(0.0))) * ind
                cnt2 = cnt2 + ind
                sp2 = sp2 + spv
                s12 = s12 + spv * tf
            return cnt2, sp2, s12

        return lax.fori_loop(0, W // 16, inner, carry)

    z = jnp.zeros((16,), jnp.float32)
    carry = (z, z, z)
    pending = issue(0)
    for k in range(NCH):
        nxt = issue(k + 1) if k + 1 < NCH else None
        for h in pending:
            h.wait()
        carry = compute(k % 2, carry)
        pending = nxt
    bacc[0] = carry[0]
    bacc[1] = carry[1]
    bacc[2] = carry[2]
    pltpu.sync_copy(bacc, oacc_hbm.at[wid])


def _key(d):
    bits = lax.bitcast_convert_type(d, jnp.int32)
    return jnp.where(bits >= 0, bits, bits ^ jnp.int32(0x7FFFFFFF))


def _bisect_body(x_ref, t_ref, w_ref, cnt_ref, sum_ref, sm_ref, sf_ref):
    j, b, h = pl.program_id(0), pl.program_id(1), pl.program_id(2)
    first = jnp.logical_and(b == 0, h == 0)

    @pl.when(jnp.logical_and(first, j == 0))
    def _():
        sm_ref[0] = jnp.int32(-2147483648)
        sm_ref[1] = 0
        sm_ref[2] = 0
        sf_ref[0] = 0.0

    @pl.when(jnp.logical_and(first, jnp.logical_and(j > 0, j <= 32)))
    def _():
        bump = jnp.where(
            sm_ref[1] <= K_RANK,
            lax.shift_left(jnp.int32(1), jnp.clip(32 - j, 0, 31)), 0)
        sm_ref[0] += bump
        sm_ref[1] = 0

    d, tt = _dt(x_ref, t_ref)
    key = _key(d)

    @pl.when(j < 32)
    def _():
        t = sm_ref[0] + lax.shift_left(jnp.int32(1), jnp.clip(31 - j, 0, 31))
        sm_ref[1] += jnp.sum((key < t).astype(jnp.int32))

    @pl.when(j == 32)
    def _():
        sel = key < sm_ref[0]
        sm_ref[2] += jnp.sum(sel.astype(jnp.int32))
        sf_ref[0] += jnp.sum(jnp.where(sel, _loss(d, tt, w_ref), 0.0))

    @pl.when(j == 33)
    def _():
        cnt_ref[0, 0] = sm_ref[2]
        sum_ref[0, 0] = sf_ref[0]


def _mk_specs(three_d):
    off = 1 if three_d else 0
    rows = ROWS_BIS if three_d else ROWS

    def xmap(*ids):
        return (ids[off], ids[off + 1], 0)

    def tmap(*ids):
        return (ids[off] * (GH_FULL if three_d else GH) + ids[off + 1], 0)

    return [
        pl.BlockSpec((2, rows, W), xmap),
        pl.BlockSpec((rows, W), tmap),
        pl.BlockSpec(memory_space=pltpu.SMEM),
    ]


def _scalar_outs(dtypes):
    return (
        tuple(jax.ShapeDtypeStruct((1, 1), dt) for dt in dtypes),
        tuple(pl.BlockSpec(memory_space=pltpu.SMEM) for _ in dtypes),
    )


@jax.jit
def kernel(predict, target, weight):
    xv = predict.reshape(B * C, H, W)
    tv = target.astype(jnp.int32).reshape(B * H, W)
    wv = weight.astype(jnp.float32)
    sc_acc = pl.kernel(
        _sc_body,
        out_type=jax.ShapeDtypeStruct((NW, 3, 16), jnp.float32),
        mesh=plsc.VectorSubcoreMesh(core_axis_name="c", subcore_axis_name="s"),
        scratch_types=[pltpu.VMEM((2, CHR, W), jnp.float32),
                       pltpu.VMEM((2, CHR, W), jnp.float32),
                       pltpu.VMEM((2, CHR, W), jnp.int32),
                       pltpu.VMEM((3, 16), jnp.float32),
                       pltpu.SemaphoreType.DMA,
                       pltpu.SemaphoreType.DMA],
    )(xv, tv)

    out_shape, out_specs = _scalar_outs((jnp.float32, jnp.float32))
    clt, s_lt = pl.pallas_call(
        _pass1_body,
        grid=(GB, GH),
        in_specs=_mk_specs(False),
        out_specs=list(out_specs),
        out_shape=list(out_shape),
        scratch_shapes=[pltpu.VMEM((ROWS, W), jnp.float32)] * 2,
    )(xv, tv, wv)

    parts = jnp.sum(sc_acc, axis=(0, 2))
    cnt_lt = (clt[0, 0] + parts[0]).astype(jnp.int32)
    s_all = s_lt[0, 0] + wv[0] * (parts[1] - parts[2]) + wv[1] * parts[2]

    def common(_):
        return s_all, cnt_lt

    def rare(_):
        o_shape, o_specs = _scalar_outs((jnp.int32, jnp.float32))
        cnt, tot = pl.pallas_call(
            _bisect_body,
            grid=(34, GB, GH_FULL),
            in_specs=_mk_specs(True),
            out_specs=list(o_specs),
            out_shape=list(o_shape),
            scratch_shapes=[pltpu.SMEM((3,), jnp.int32),
                            pltpu.SMEM((1,), jnp.float32)],
        )(xv, tv, wv)
        return tot[0, 0], cnt[0, 0]

    total, cnt = lax.cond(cnt_lt >= K_RANK + 1, common, rare, operand=None)
    return jnp.where(cnt == 0, total,
                     total / jnp.maximum(cnt, 1).astype(jnp.float32))


# speedup vs baseline: 2.0996x; 1.0049x over previous
import jax
import jax.numpy as jnp
from jax import lax
from jax.experimental import pallas as pl
from jax.experimental.pallas import tpu as pltpu
from jax.experimental.pallas import tpu_sc as plsc

B, C, H, W = 8, 2, 512, 512
HW = H * W
N = B * H * W
MIN_KEPT = 100000
K_RANK = min(MIN_KEPT, N - 1)
LOGIT_T = 0.8472978603872037

H_TC = 352
ROWS = 176
GB, GH = B, H_TC // ROWS
ROWS_BIS = 128
GH_FULL = H // ROWS_BIS

NC, NS = 2, 16
NW = NC * NS
RPT = (H - H_TC) // (NW // B)
CHR = 8
NCH = RPT // CHR

_LOG1P_C = (0.9999970542923676, -0.4998254710554547, 0.33078789064803327,
            -0.23417367475167797, 0.14810677481238943, -0.06577012721513113,
            0.014026852411466048)


def _dt(x_ref, t_ref):
    diff = x_ref[1] - x_ref[0]
    tt = t_ref[...]
    return jnp.where(tt == 1, diff, -diff), tt


def _loss(d, tt, w_ref):
    wt = jnp.where(tt == 1, w_ref[1], w_ref[0])
    return wt * (jnp.log1p(jnp.exp(-jnp.abs(d))) + jnp.maximum(-d, 0.0))


def _pass1_body(x_ref, t_ref, w_ref, clt_ref, sum_ref, a_lt, a_sum):
    b, h = pl.program_id(0), pl.program_id(1)
    first = jnp.logical_and(b == 0, h == 0)
    last = jnp.logical_and(b == GB - 1, h == GH - 1)

    @pl.when(first)
    def _():
        a_lt[...] = jnp.zeros_like(a_lt)
        a_sum[...] = jnp.zeros_like(a_sum)

    d, tt = _dt(x_ref, t_ref)
    L = jnp.float32(LOGIT_T)
    sel = d < L
    one = jnp.float32(1.0)
    zero = jnp.float32(0.0)
    a_lt[...] += jnp.where(sel, one, zero)
    a_sum[...] += jnp.where(sel, _loss(d, tt, w_ref), zero)

    @pl.when(last)
    def _():
        clt_ref[0, 0] = jnp.sum(a_lt[...])
        sum_ref[0, 0] = jnp.sum(a_sum[...])


def _sc_body(x_hbm, t_hbm, oacc_hbm, bx0, bx1, bt, bacc, sem0, sem1):
    cc = lax.axis_index("c")
    ss = lax.axis_index("s")
    wid = ss * NC + cc
    b = wid // (NW // B)
    q = wid % (NW // B)
    row0 = H_TC + q * RPT

    L = jnp.float32(LOGIT_T)
    sems = (sem0, sem1)

    def issue(k):
        r = row0 + k * CHR
        p = k % 2
        s = sems[p]
        return (
            pltpu.async_copy(x_hbm.at[2 * b, pl.ds(r, CHR), :], bx0.at[p], s),
            pltpu.async_copy(x_hbm.at[2 * b + 1, pl.ds(r, CHR), :],
                             bx1.at[p], s),
            pltpu.async_copy(t_hbm.at[pl.ds(b * H + r, CHR), :], bt.at[p], s),
        )

    def compute(p, carry):
        def inner(i, c2):
            cnt2, sp2, s12 = c2
            s16 = pl.ds(i * 16, 16)
            for u in range(CHR):
                x0 = bx0[p, u, s16]
                x1 = bx1[p, u, s16]
                tf = bt[p, u, s16].astype(jnp.float32)
                d = (x1 - x0) * (tf + tf - 1.0)
                ind = jnp.where(d < L, jnp.float32(1.0), jnp.float32(0.0))
                y = jnp.exp(-jnp.abs(d))
                qq = jnp.float32(_LOG1P_C[6])
                for cf in _LOG1P_C[5::-1]:
                    qq = qq * y + jnp.float32(cf)
                spv = (y * qq + jnp.maximum(-d, jnp.float32(0.0))) * ind
                cnt2 = cnt2 + ind
                sp2 = sp2 + spv
                s12 = s12 + spv * tf
            return cnt2, sp2, s12

        return lax.fori_loop(0, W // 16, inner, carry)

    z = jnp.zeros((16,), jnp.float32)
    carry = (z, z, z)
    pending = issue(0)
    for k in range(NCH):
        nxt = issue(k + 1) if k + 1 < NCH else None
        for h in pending:
            h.wait()
        carry = compute(k % 2, carry)
        pending = nxt
    bacc[0] = carry[0]
    bacc[1] = carry[1]
    bacc[2] = carry[2]
    pltpu.sync_copy(bacc, oacc_hbm.at[wid])


def _key(d):
    bits = lax.bitcast_convert_type(d, jnp.int32)
    return jnp.where(bits >= 0, bits, bits ^ jnp.int32(0x7FFFFFFF))


def _bisect_body(x_ref, t_ref, w_ref, cnt_ref, sum_ref, sm_ref, sf_ref):
    j, b, h = pl.program_id(0), pl.program_id(1), pl.program_id(2)
    first = jnp.logical_and(b == 0, h == 0)

    @pl.when(jnp.logical_and(first, j == 0))
    def _():
        sm_ref[0] = jnp.int32(-2147483648)
        sm_ref[1] = 0
        sm_ref[2] = 0
        sf_ref[0] = 0.0

    @pl.when(jnp.logical_and(first, jnp.logical_and(j > 0, j <= 32)))
    def _():
        bump = jnp.where(
            sm_ref[1] <= K_RANK,
            lax.shift_left(jnp.int32(1), jnp.clip(32 - j, 0, 31)), 0)
        sm_ref[0] += bump
        sm_ref[1] = 0

    d, tt = _dt(x_ref, t_ref)
    key = _key(d)

    @pl.when(j < 32)
    def _():
        t = sm_ref[0] + lax.shift_left(jnp.int32(1), jnp.clip(31 - j, 0, 31))
        sm_ref[1] += jnp.sum((key < t).astype(jnp.int32))

    @pl.when(j == 32)
    def _():
        sel = key < sm_ref[0]
        sm_ref[2] += jnp.sum(sel.astype(jnp.int32))
        sf_ref[0] += jnp.sum(jnp.where(sel, _loss(d, tt, w_ref), 0.0))

    @pl.when(j == 33)
    def _():
        cnt_ref[0, 0] = sm_ref[2]
        sum_ref[0, 0] = sf_ref[0]


def _mk_specs(three_d):
    off = 1 if three_d else 0
    rows = ROWS_BIS if three_d else ROWS

    def xmap(*ids):
        return (ids[off], ids[off + 1], 0)

    def tmap(*ids):
        return (ids[off] * (GH_FULL if three_d else GH) + ids[off + 1], 0)

    return [
        pl.BlockSpec((2, rows, W), xmap),
        pl.BlockSpec((rows, W), tmap),
        pl.BlockSpec(memory_space=pltpu.SMEM),
    ]


def _scalar_outs(dtypes):
    return (
        tuple(jax.ShapeDtypeStruct((1, 1), dt) for dt in dtypes),
        tuple(pl.BlockSpec(memory_space=pltpu.SMEM) for _ in dtypes),
    )


@jax.jit
def kernel(predict, target, weight):
    xv = predict.reshape(B * C, H, W)
    tv = target.astype(jnp.int32).reshape(B * H, W)
    wv = weight.astype(jnp.float32)
    sc_acc = pl.kernel(
        _sc_body,
        out_type=jax.ShapeDtypeStruct((NW, 3, 16), jnp.float32),
        mesh=plsc.VectorSubcoreMesh(core_axis_name="c", subcore_axis_name="s"),
        scratch_types=[pltpu.VMEM((2, CHR, W), jnp.float32),
                       pltpu.VMEM((2, CHR, W), jnp.float32),
                       pltpu.VMEM((2, CHR, W), jnp.int32),
                       pltpu.VMEM((3, 16), jnp.float32),
                       pltpu.SemaphoreType.DMA,
                       pltpu.SemaphoreType.DMA],
    )(xv, tv)

    out_shape, out_specs = _scalar_outs((jnp.float32, jnp.float32))
    clt, s_lt = pl.pallas_call(
        _pass1_body,
        grid=(GB, GH),
        in_specs=_mk_specs(False),
        out_specs=list(out_specs),
        out_shape=list(out_shape),
        scratch_shapes=[pltpu.VMEM((ROWS, W), jnp.float32)] * 2,
    )(xv, tv, wv)

    parts = jnp.sum(sc_acc, axis=(0, 2))
    cnt_lt = (clt[0, 0] + parts[0]).astype(jnp.int32)
    s_all = s_lt[0, 0] + wv[0] * (parts[1] - parts[2]) + wv[1] * parts[2]

    def common(_):
        return s_all, cnt_lt

    def rare(_):
        o_shape, o_specs = _scalar_outs((jnp.int32, jnp.float32))
        cnt, tot = pl.pallas_call(
            _bisect_body,
            grid=(34, GB, GH_FULL),
            in_specs=_mk_specs(True),
            out_specs=list(o_specs),
            out_shape=list(o_shape),
            scratch_shapes=[pltpu.SMEM((3,), jnp.int32),
                            pltpu.SMEM((1,), jnp.float32)],
        )(xv, tv, wv)
        return tot[0, 0], cnt[0, 0]

    total, cnt = lax.cond(cnt_lt >= K_RANK + 1, common, rare, operand=None)
    return jnp.where(cnt == 0, total,
                     total / jnp.maximum(cnt, 1).astype(jnp.float32))
